# baseline (device time: 150043 ns/iter reference)
import jax
import jax.numpy as jnp
from jax import lax
from jax.experimental import pallas as pl
from jax.experimental.pallas import tpu as pltpu

N_DEV = 4


def kernel(A, B):
    M, K = A.shape
    _, N = B.shape

    A = A.astype(jnp.bfloat16)
    B = B.astype(jnp.bfloat16)

    def body(a_ref, b_ref, out_ref, comm_ref, send_sems, recv_sems):
        my_pos = lax.axis_index("i")
        left = (my_pos - 1) % N_DEV
        right = (my_pos + 1) % N_DEV

        barrier_sem = pltpu.get_barrier_semaphore()
        for nbr in [left, right]:
            pl.semaphore_signal(
                barrier_sem, inc=1,
                device_id=(nbr,), device_id_type=pl.DeviceIdType.MESH,
            )
        pl.semaphore_wait(barrier_sem, 2)

        partial = jnp.dot(a_ref[...], b_ref[...],
                          preferred_element_type=jnp.float32)
        out_ref[...] = partial
        comm_ref[0, :, :] = partial

        for h in range(N_DEV - 1):
            send_slot = h % 2
            recv_slot = (h + 1) % 2
            rdma = pltpu.make_async_remote_copy(
                src_ref=comm_ref.at[send_slot],
                dst_ref=comm_ref.at[recv_slot],
                send_sem=send_sems.at[send_slot],
                recv_sem=recv_sems.at[recv_slot],
                device_id=(right,),
                device_id_type=pl.DeviceIdType.MESH,
            )
            rdma.start()
            rdma.wait()
            out_ref[...] += comm_ref[recv_slot, :, :]

    return pl.pallas_call(
        body,
        out_shape=jax.ShapeDtypeStruct((M, N), jnp.float32),
        in_specs=[
            pl.BlockSpec(memory_space=pltpu.VMEM),
            pl.BlockSpec(memory_space=pltpu.VMEM),
        ],
        out_specs=pl.BlockSpec(memory_space=pltpu.VMEM),
        scratch_shapes=[
            pltpu.VMEM((2, M, N), jnp.float32),
            pltpu.SemaphoreType.DMA((2,)),
            pltpu.SemaphoreType.DMA((2,)),
        ],
        compiler_params=pltpu.CompilerParams(collective_id=0),
    )(A, B)


# device time: 32902 ns/iter; 4.5603x vs baseline; 4.5603x over previous
import jax
import jax.numpy as jnp
from jax import lax
from jax.experimental import pallas as pl
from jax.experimental.pallas import tpu as pltpu

N_DEV = 4
M = 1024
N = 1024
NH = N // 2
RH = M // 2
RQ = M // 4


def kernel(A, B):
    A = A.astype(jnp.bfloat16)
    B = B.astype(jnp.bfloat16)

    def body(a_ref, b_ref, out_ref, sbuf_h, rbuf_h, sbuf_q, rbuf_q,
             send_sems, recv_sems):
        p = lax.axis_index("i")
        pA = p ^ 1
        pB = 3 - p

        barrier_sem = pltpu.get_barrier_semaphore()
        for nbr in [pA, pB]:
            pl.semaphore_signal(
                barrier_sem, inc=1,
                device_id=(nbr,), device_id_type=pl.DeviceIdType.MESH,
            )
        pl.semaphore_wait(barrier_sem, 2)

        out_ref[...] = jnp.dot(a_ref[...], b_ref[...],
                               preferred_element_type=jnp.float32)

        a_keep = (p ^ (p >> 1)) & 1
        b_keep = p >> 1
        h = [a_keep * RH, b_keep * RH]
        q = [(p >> 1) * RQ, (p & 1) * RQ]
        partner1 = [pA, pB]
        partner2 = [pB, pA]

        def exchange(step, tree, partner, sbuf, rbuf):
            idx = 2 * step + tree
            return pltpu.make_async_remote_copy(
                src_ref=sbuf.at[tree],
                dst_ref=rbuf.at[tree],
                send_sem=send_sems.at[idx],
                recv_sem=recv_sems.at[idx],
                device_id=(partner[tree],),
                device_id_type=pl.DeviceIdType.MESH,
            )

        cols = [slice(0, NH), slice(NH, N)]

        rdmas = []
        for t in range(2):
            send_rows = pl.ds((1 - [a_keep, b_keep][t]) * RH, RH)
            sbuf_h[t] = out_ref[send_rows, cols[t]].astype(jnp.bfloat16)
            r = exchange(0, t, partner1, sbuf_h, rbuf_h)
            r.start()
            rdmas.append(r)
        for t in range(2):
            rdmas[t].wait()
            keep_rows = pl.ds(h[t], RH)
            out_ref[keep_rows, cols[t]] += rbuf_h[t].astype(jnp.float32)

        rdmas = []
        for t in range(2):
            send_rows = pl.ds(h[t] + (RQ - q[t]), RQ)
            sbuf_q[t] = out_ref[send_rows, cols[t]].astype(jnp.bfloat16)
            r = exchange(1, t, partner2, sbuf_q, rbuf_q)
            r.start()
            rdmas.append(r)
        for t in range(2):
            rdmas[t].wait()
            keep_rows = pl.ds(h[t] + q[t], RQ)
            out_ref[keep_rows, cols[t]] += rbuf_q[t].astype(jnp.float32)

        rdmas = []
        for t in range(2):
            own_rows = pl.ds(h[t] + q[t], RQ)
            sbuf_q[t] = out_ref[own_rows, cols[t]].astype(jnp.bfloat16)
            r = exchange(2, t, partner2, sbuf_q, rbuf_q)
            r.start()
            rdmas.append(r)
        for t in range(2):
            rdmas[t].wait()
            other_rows = pl.ds(h[t] + (RQ - q[t]), RQ)
            out_ref[other_rows, cols[t]] = rbuf_q[t].astype(jnp.float32)

        rdmas = []
        for t in range(2):
            own_rows = pl.ds(h[t], RH)
            sbuf_h[t] = out_ref[own_rows, cols[t]].astype(jnp.bfloat16)
            r = exchange(3, t, partner1, sbuf_h, rbuf_h)
            r.start()
            rdmas.append(r)
        for t in range(2):
            rdmas[t].wait()
            other_rows = pl.ds(RH - h[t], RH)
            out_ref[other_rows, cols[t]] = rbuf_h[t].astype(jnp.float32)

    return pl.pallas_call(
        body,
        out_shape=jax.ShapeDtypeStruct((M, N), jnp.float32),
        in_specs=[
            pl.BlockSpec(memory_space=pltpu.VMEM),
            pl.BlockSpec(memory_space=pltpu.VMEM),
        ],
        out_specs=pl.BlockSpec(memory_space=pltpu.VMEM),
        scratch_shapes=[
            pltpu.VMEM((2, RH, NH), jnp.bfloat16),
            pltpu.VMEM((2, RH, NH), jnp.bfloat16),
            pltpu.VMEM((2, RQ, NH), jnp.bfloat16),
            pltpu.VMEM((2, RQ, NH), jnp.bfloat16),
            pltpu.SemaphoreType.DMA((8,)),
            pltpu.SemaphoreType.DMA((8,)),
        ],
        compiler_params=pltpu.CompilerParams(collective_id=0),
    )(A, B)


# device time: 30469 ns/iter; 4.9244x vs baseline; 1.0799x over previous
import jax
import jax.numpy as jnp
from jax import lax
from jax.experimental import pallas as pl
from jax.experimental.pallas import tpu as pltpu

N_DEV = 4
M = 1024
N = 1024
T = M // 2
H = T // 2
Q = H // 2


def kernel(A, B):
    A = A.astype(jnp.bfloat16)
    B = B.astype(jnp.bfloat16)

    def body(a_ref, b_ref, out_ref, p_ref, sbuf_h, rbuf_h, sbuf_q, rbuf_q,
             send_sems, recv_sems):
        p = lax.axis_index("i")
        pA = p ^ 1
        pB = 3 - p

        barrier_sem = pltpu.get_barrier_semaphore()
        for nbr in [pA, pB]:
            pl.semaphore_signal(
                barrier_sem, inc=1,
                device_id=(nbr,), device_id_type=pl.DeviceIdType.MESH,
            )
        pl.semaphore_wait(barrier_sem, 2)

        keep = [(p ^ (p >> 1)) & 1, p >> 1]
        qoff = [(p >> 1) * Q, (p & 1) * Q]
        part1 = [pA, pB]
        part2 = [pB, pA]

        hs = [t * T + keep[t] * H for t in range(2)]
        os_ = [t * T + (1 - keep[t]) * H for t in range(2)]
        qs = [hs[t] + qoff[t] for t in range(2)]

        def xchg(step, t, partner, src, dst):
            idx = 2 * step + t
            return pltpu.make_async_remote_copy(
                src_ref=src, dst_ref=dst,
                send_sem=send_sems.at[idx], recv_sem=recv_sems.at[idx],
                device_id=(partner[t],), device_id_type=pl.DeviceIdType.MESH,
            )

        rd0 = []
        for t in range(2):
            sbuf_h[t] = jnp.dot(
                a_ref[pl.ds(os_[t], H), :], b_ref[...],
                preferred_element_type=jnp.float32,
            ).astype(jnp.bfloat16)
            r = xchg(0, t, part1, sbuf_h.at[t], rbuf_h.at[t])
            r.start()
            rd0.append(r)
        for t in range(2):
            p_ref[pl.ds(hs[t], H), :] = jnp.dot(
                a_ref[pl.ds(hs[t], H), :], b_ref[...],
                preferred_element_type=jnp.float32,
            )

        rd1 = []
        for t in range(2):
            rd0[t].wait()
            ro = Q - qoff[t]
            sbuf_q[t] = (
                p_ref[pl.ds(hs[t] + ro, Q), :]
                + rbuf_h[t, pl.ds(ro, Q), :].astype(jnp.float32)
            ).astype(jnp.bfloat16)
            r = xchg(1, t, part2, sbuf_q.at[t], rbuf_q.at[t])
            r.start()
            rd1.append(r)
        for t in range(2):
            p_ref[pl.ds(qs[t], Q), :] += (
                rbuf_h[t, pl.ds(qoff[t], Q), :].astype(jnp.float32)
            )

        rd2 = []
        for t in range(2):
            rd1[t].wait()
            out_ref[pl.ds(qs[t], Q), :] = (
                p_ref[pl.ds(qs[t], Q), :]
                + rbuf_q[t].astype(jnp.float32)
            ).astype(jnp.bfloat16)
            r = xchg(2, t, part2,
                     out_ref.at[pl.ds(qs[t], Q)],
                     out_ref.at[pl.ds(qs[t], Q)])
            r.start()
            rd2.append(r)

        rd3 = []
        for t in range(2):
            rd2[t].wait()
            r = xchg(3, t, part1,
                     out_ref.at[pl.ds(hs[t], H)],
                     out_ref.at[pl.ds(hs[t], H)])
            r.start()
            rd3.append(r)
        for t in range(2):
            rd3[t].wait()

    return pl.pallas_call(
        body,
        out_shape=jax.ShapeDtypeStruct((M, N), jnp.bfloat16),
        in_specs=[
            pl.BlockSpec(memory_space=pltpu.VMEM),
            pl.BlockSpec(memory_space=pltpu.VMEM),
        ],
        out_specs=pl.BlockSpec(memory_space=pltpu.VMEM),
        scratch_shapes=[
            pltpu.VMEM((M, N), jnp.float32),
            pltpu.VMEM((2, H, N), jnp.bfloat16),
            pltpu.VMEM((2, H, N), jnp.bfloat16),
            pltpu.VMEM((2, Q, N), jnp.bfloat16),
            pltpu.VMEM((2, Q, N), jnp.bfloat16),
            pltpu.SemaphoreType.DMA((8,)),
            pltpu.SemaphoreType.DMA((8,)),
        ],
        compiler_params=pltpu.CompilerParams(collective_id=0),
    )(A, B)


# device time: 28397 ns/iter; 5.2838x vs baseline; 1.0730x over previous
import jax
import jax.numpy as jnp
from jax import lax
from jax.experimental import pallas as pl
from jax.experimental.pallas import tpu as pltpu

N_DEV = 4
M = 1024
K = 512
N = 1024
T = M // 2
H = T // 2
Q = H // 2


def kernel(A, B):
    def body(a_ref, b_ref, out_ref, p_ref, b16_ref, sbuf_h, rbuf_h,
             sbuf_q, rbuf_q, send_sems, recv_sems):
        p = lax.axis_index("i")
        pA = p ^ 1
        pB = 3 - p

        barrier_sem = pltpu.get_barrier_semaphore()
        for nbr in [pA, pB]:
            pl.semaphore_signal(
                barrier_sem, inc=1,
                device_id=(nbr,), device_id_type=pl.DeviceIdType.MESH,
            )
        b16_ref[...] = b_ref[...].astype(jnp.bfloat16)
        pl.semaphore_wait(barrier_sem, 2)

        keep = [(p ^ (p >> 1)) & 1, p >> 1]
        qoff = [(p >> 1) * Q, (p & 1) * Q]
        part1 = [pA, pB]
        part2 = [pB, pA]

        hs = [t * T + keep[t] * H for t in range(2)]
        os_ = [t * T + (1 - keep[t]) * H for t in range(2)]
        qs = [hs[t] + qoff[t] for t in range(2)]
        ro = [Q - qoff[t] for t in range(2)]
        foff = [Q - qoff[0], qoff[1]]

        def xchg(idx, partner, src, dst):
            return pltpu.make_async_remote_copy(
                src_ref=src, dst_ref=dst,
                send_sem=send_sems.at[idx], recv_sem=recv_sems.at[idx],
                device_id=(partner,), device_id_type=pl.DeviceIdType.MESH,
            )

        def qdot(row_start):
            a = a_ref[pl.ds(row_start, Q), :].astype(jnp.bfloat16)
            return jnp.dot(a, b16_ref[...],
                           preferred_element_type=jnp.float32)

        rd0a, rd0b = [], []
        for t in range(2):
            f = foff[t]
            sbuf_h[t, pl.ds(f, Q), :] = qdot(os_[t] + f).astype(jnp.bfloat16)
            r = xchg(0 + t, part1[t],
                     sbuf_h.at[t, pl.ds(f, Q)], rbuf_h.at[t, pl.ds(f, Q)])
            r.start()
            rd0a.append(r)
        for t in range(2):
            g = Q - foff[t]
            sbuf_h[t, pl.ds(g, Q), :] = qdot(os_[t] + g).astype(jnp.bfloat16)
            r = xchg(2 + t, part1[t],
                     sbuf_h.at[t, pl.ds(g, Q)], rbuf_h.at[t, pl.ds(g, Q)])
            r.start()
            rd0b.append(r)
        for t in range(2):
            p_ref[pl.ds(hs[t], Q), :] = qdot(hs[t])
            p_ref[pl.ds(hs[t] + Q, Q), :] = qdot(hs[t] + Q)

        rd1 = []
        for t in range(2):
            rd0a[t].wait()
            sbuf_q[t] = (
                p_ref[pl.ds(hs[t] + ro[t], Q), :]
                + rbuf_h[t, pl.ds(ro[t], Q), :].astype(jnp.float32)
            ).astype(jnp.bfloat16)
            r = xchg(4 + t, part2[t], sbuf_q.at[t], rbuf_q.at[t])
            r.start()
            rd1.append(r)
        for t in range(2):
            rd0b[t].wait()
            p_ref[pl.ds(qs[t], Q), :] += (
                rbuf_h[t, pl.ds(qoff[t], Q), :].astype(jnp.float32)
            )

        rd2, rd3a = [], []
        for t in range(2):
            rd1[t].wait()
            out_ref[pl.ds(qs[t], Q), :] = (
                p_ref[pl.ds(qs[t], Q), :]
                + rbuf_q[t].astype(jnp.float32)
            ).astype(jnp.bfloat16)
            r = xchg(6 + t, part2[t],
                     out_ref.at[pl.ds(qs[t], Q)],
                     out_ref.at[pl.ds(qs[t], Q)])
            r.start()
            rd2.append(r)
        for t in range(2):
            r = xchg(8 + t, part1[t],
                     out_ref.at[pl.ds(qs[t], Q)],
                     out_ref.at[pl.ds(qs[t], Q)])
            r.start()
            rd3a.append(r)

        rd3b = []
        for t in range(2):
            rd2[t].wait()
            r = xchg(10 + t, part1[t],
                     out_ref.at[pl.ds(hs[t] + ro[t], Q)],
                     out_ref.at[pl.ds(hs[t] + ro[t], Q)])
            r.start()
            rd3b.append(r)
        for t in range(2):
            rd3a[t].wait()
            rd3b[t].wait()

    return pl.pallas_call(
        body,
        out_shape=jax.ShapeDtypeStruct((M, N), jnp.bfloat16),
        in_specs=[
            pl.BlockSpec(memory_space=pltpu.VMEM),
            pl.BlockSpec(memory_space=pltpu.VMEM),
        ],
        out_specs=pl.BlockSpec(memory_space=pltpu.VMEM),
        scratch_shapes=[
            pltpu.VMEM((M, N), jnp.float32),
            pltpu.VMEM((K, N), jnp.bfloat16),
            pltpu.VMEM((2, H, N), jnp.bfloat16),
            pltpu.VMEM((2, H, N), jnp.bfloat16),
            pltpu.VMEM((2, Q, N), jnp.bfloat16),
            pltpu.VMEM((2, Q, N), jnp.bfloat16),
            pltpu.SemaphoreType.DMA((12,)),
            pltpu.SemaphoreType.DMA((12,)),
        ],
        compiler_params=pltpu.CompilerParams(collective_id=0),
    )(A, B)


# device time: 27032 ns/iter; 5.5506x vs baseline; 1.0505x over previous
import jax
import jax.numpy as jnp
from jax import lax
from jax.experimental import pallas as pl
from jax.experimental.pallas import tpu as pltpu

N_DEV = 4
M = 1024
K = 512
N = 1024
T = M // 2
H = T // 2
S = H // 2


def kernel(A, B):
    def body(a_ref, b_ref, out_ref, p_ref, b16_ref, sbuf0, rbuf0,
             sbuf1, rbuf1, send_sems, recv_sems):
        p = lax.axis_index("i")
        pA = p ^ 1
        pB = 3 - p

        barrier_sem = pltpu.get_barrier_semaphore()
        for nbr in [pA, pB]:
            pl.semaphore_signal(
                barrier_sem, inc=1,
                device_id=(nbr,), device_id_type=pl.DeviceIdType.MESH,
            )
        b16_ref[...] = b_ref[...].astype(jnp.bfloat16)
        pl.semaphore_wait(barrier_sem, 2)

        keep = [(p ^ (p >> 1)) & 1, p >> 1]
        part1 = [pA, pB]
        part2 = [pB, pA]

        hs = [t * T + keep[t] * H for t in range(2)]
        os_ = [t * T + (1 - keep[t]) * H for t in range(2)]

        def xchg(idx, partner, src, dst):
            return pltpu.make_async_remote_copy(
                src_ref=src, dst_ref=dst,
                send_sem=send_sems.at[idx], recv_sem=recv_sems.at[idx],
                device_id=(partner,), device_id_type=pl.DeviceIdType.MESH,
            )

        def dot_rows(row_start, nrows):
            a = a_ref[pl.ds(row_start, nrows), :].astype(jnp.bfloat16)
            return jnp.dot(a, b16_ref[...],
                           preferred_element_type=jnp.float32)

        rd0 = [[None, None], [None, None]]
        for j in range(2):
            for t in range(2):
                sbuf0[t, pl.ds(j * S, S), :] = (
                    dot_rows(os_[t] + j * S, S).astype(jnp.bfloat16)
                )
                r = xchg(2 * j + t, part1[t],
                         sbuf0.at[t, pl.ds(j * S, S)],
                         rbuf0.at[t, pl.ds(j * S, S)])
                r.start()
                rd0[t][j] = r
        for t in range(2):
            p_ref[pl.ds(hs[t], H), :] = dot_rows(hs[t], H)

        rd1 = [[None, None], [None, None]]
        for j in range(2):
            for t in range(2):
                rd0[t][j].wait()
                rows = pl.ds(hs[t] + j * S, S)
                acc = (p_ref[rows, :]
                       + rbuf0[t, pl.ds(j * S, S), :].astype(jnp.float32))
                p_ref[rows, :] = acc
                sbuf1[t, pl.ds(j * S, S), :] = acc.astype(jnp.bfloat16)
                r = xchg(4 + 2 * j + t, part2[t],
                         sbuf1.at[t, pl.ds(j * S, S)],
                         rbuf1.at[t, pl.ds(j * S, S)])
                r.start()
                rd1[t][j] = r

        rd2 = [[None, None], [None, None]]
        for j in range(2):
            for t in range(2):
                rd1[t][j].wait()
                rows = pl.ds(hs[t] + j * S, S)
                out_ref[rows, :] = (
                    p_ref[rows, :]
                    + rbuf1[t, pl.ds(j * S, S), :].astype(jnp.float32)
                ).astype(jnp.bfloat16)
                r = xchg(8 + 2 * j + t, part1[t],
                         out_ref.at[rows], out_ref.at[rows])
                r.start()
                rd2[t][j] = r
        for t in range(2):
            rd2[t][0].wait()
            rd2[t][1].wait()

    return pl.pallas_call(
        body,
        out_shape=jax.ShapeDtypeStruct((M, N), jnp.bfloat16),
        in_specs=[
            pl.BlockSpec(memory_space=pltpu.VMEM),
            pl.BlockSpec(memory_space=pltpu.VMEM),
        ],
        out_specs=pl.BlockSpec(memory_space=pltpu.VMEM),
        scratch_shapes=[
            pltpu.VMEM((M, N), jnp.float32),
            pltpu.VMEM((K, N), jnp.bfloat16),
            pltpu.VMEM((2, H, N), jnp.bfloat16),
            pltpu.VMEM((2, H, N), jnp.bfloat16),
            pltpu.VMEM((2, H, N), jnp.bfloat16),
            pltpu.VMEM((2, H, N), jnp.bfloat16),
            pltpu.SemaphoreType.DMA((12,)),
            pltpu.SemaphoreType.DMA((12,)),
        ],
        compiler_params=pltpu.CompilerParams(collective_id=0),
    )(A, B)
